# baseline (speedup 1.0000x reference)
import functools

import jax
import jax.numpy as jnp
from jax import lax
from jax.experimental import pallas as pl
from jax.experimental.pallas import tpu as pltpu
from jax.experimental.pallas import tpu_sc as plsc

D_MODEL = 128
MAX_LEN = 100
N_EDGES = 320000

_NUM_CORES = 2
_NUM_SUBCORES = 16
_NW = _NUM_CORES * _NUM_SUBCORES
_B_PER_W = N_EDGES // _NW

_ROUNDS = 10
_NBA = 4
_CHA = 216
_CHB = 136
assert _ROUNDS * (_NBA * _CHA + _CHB) == _B_PER_W
_NA = _NBA * _ROUNDS
_A_SPAN = _NA * _CHA

_mesh = plsc.VectorSubcoreMesh(core_axis_name="c", subcore_axis_name="s")


@functools.partial(
    pl.kernel,
    mesh=_mesh,
    out_type=jax.ShapeDtypeStruct((N_EDGES, D_MODEL), jnp.float32),
    scratch_types=(
        [pltpu.VMEM((_CHA,), jnp.int32) for _ in range(_NBA)]
        + [pltpu.VMEM((_CHA, D_MODEL), jnp.float32) for _ in range(_NBA)]
        + [pltpu.VMEM((_CHB,), jnp.int32)]
        + [pltpu.VMEM((_CHB, D_MODEL), jnp.float32)]
        + [pltpu.VMEM_SHARED((MAX_LEN, D_MODEL), jnp.float32)]
        + [pltpu.SemaphoreType.DMA for _ in range(3 * _NBA + 3)]
    ),
)
def _pe_gather(idx_hbm, table_hbm, out_hbm, *refs):
    idxa = refs[0:_NBA]
    rowsa = refs[_NBA:2 * _NBA]
    idxb = refs[2 * _NBA]
    rowsb = refs[2 * _NBA + 1]
    table_v = refs[2 * _NBA + 2]
    sems = refs[2 * _NBA + 3:]
    sia = sems[0:_NBA]
    sga = sems[_NBA:2 * _NBA]
    swa = sems[2 * _NBA:3 * _NBA]
    sib = sems[3 * _NBA]
    sgb = sems[3 * _NBA + 1]
    swb = sems[3 * _NBA + 2]

    wid = lax.axis_index("s") * _NUM_CORES + lax.axis_index("c")
    base = wid * _B_PER_W
    baseb = base + _A_SPAN

    @pl.when(lax.axis_index("s") == 0)
    def _stage_table():
        pltpu.sync_copy(table_hbm, table_v)

    plsc.subcore_barrier()

    def fire_ia(slot, off):
        pltpu.async_copy(idx_hbm.at[pl.ds(off, _CHA)], idxa[slot], sia[slot])

    def wait_ia(slot, off):
        pltpu.make_async_copy(
            idx_hbm.at[pl.ds(off, _CHA)], idxa[slot], sia[slot]).wait()

    def fire_ga(slot):
        pltpu.async_copy(table_v.at[idxa[slot]], rowsa[slot], sga[slot])

    def wait_ga(slot):
        pltpu.make_async_copy(
            table_v.at[idxa[slot]], rowsa[slot], sga[slot]).wait()

    def fire_wa(slot, off):
        pltpu.async_copy(rowsa[slot], out_hbm.at[pl.ds(off, _CHA)], swa[slot])

    def wait_wa(slot, off):
        pltpu.make_async_copy(
            rowsa[slot], out_hbm.at[pl.ds(off, _CHA)], swa[slot]).wait()

    def fire_ib(off):
        pltpu.async_copy(idx_hbm.at[pl.ds(off, _CHB)], idxb, sib)

    def wait_ib(off):
        pltpu.make_async_copy(
            idx_hbm.at[pl.ds(off, _CHB)], idxb, sib).wait()

    def fire_gb():
        pltpu.async_copy(table_hbm.at[idxb], rowsb, sgb)

    def wait_gb():
        pltpu.make_async_copy(table_hbm.at[idxb], rowsb, sgb).wait()

    def fire_wb(off):
        pltpu.async_copy(rowsb, out_hbm.at[pl.ds(off, _CHB)], swb)

    def wait_wb(off):
        pltpu.make_async_copy(rowsb, out_hbm.at[pl.ds(off, _CHB)], swb).wait()

    for b in range(_NBA):
        fire_ia(b, base + b * _CHA)
    fire_ib(baseb)

    wait_ia(0, base)
    fire_ga(0)
    wait_ia(1, base + _CHA)
    fire_ga(1)
    wait_ga(0)
    fire_wa(0, base)
    fire_ia(0, base + _NBA * _CHA)
    wait_ib(baseb)
    fire_gb()
    for c in (2, 3):
        b, bp = c % _NBA, (c - 1) % _NBA
        wait_ia(b, base + c * _CHA)
        fire_ga(b)
        wait_ga(bp)
        fire_wa(bp, base + (c - 1) * _CHA)
        fire_ia(bp, base + (c + _NBA - 1) * _CHA)

    def round_(k, carry):
        wait_gb()
        fire_wb(baseb + (k - 1) * _CHB)
        fire_ib(baseb + k * _CHB)

        for b in range(_NBA):
            bp = (b - 1) % _NBA
            c = k * _NBA + b
            off = base + c * _CHA
            wait_ia(b, off)
            wait_wa(b, off - _NBA * _CHA)
            fire_ga(b)
            wait_ga(bp)
            fire_wa(bp, off - _CHA)
            p = jnp.where(c + _NBA - 1 < _NA, c + _NBA - 1, c - 1)
            fire_ia(bp, base + p * _CHA)
            if b == 1:
                wait_wb(baseb + (k - 1) * _CHB)
                wait_ib(baseb + k * _CHB)
                fire_gb()
        return carry

    lax.fori_loop(1, _ROUNDS, round_, 0)

    last = _NA - 1
    bl = last % _NBA
    wait_ga(bl)
    fire_wa(bl, base + last * _CHA)
    wait_gb()
    fire_wb(baseb + (_ROUNDS - 1) * _CHB)
    for c in range(_NA - _NBA, _NA):
        wait_wa(c % _NBA, base + c * _CHA)
    wait_wb(baseb + (_ROUNDS - 1) * _CHB)
    for c in range(_NA - _NBA + 1, _NA):
        wait_ia((c - 1) % _NBA, base + (c - 1) * _CHA)


def kernel(edge_type, pe):
    return _pe_gather(edge_type.astype(jnp.int32), pe)


# speedup vs baseline: 2.0790x; 2.0790x over previous
import functools

import jax
import jax.numpy as jnp
from jax import lax
from jax.experimental import pallas as pl
from jax.experimental.pallas import tpu as pltpu
from jax.experimental.pallas import tpu_sc as plsc

D_MODEL = 128
MAX_LEN = 100
N_EDGES = 320000

_NUM_CORES = 2
_NUM_SUBCORES = 16
_NW = _NUM_CORES * _NUM_SUBCORES
_B_PER_W = N_EDGES // _NW
_CH = 200
_NCH = _B_PER_W // _CH
_NBUF = 5
_LAG = 2

_mesh = plsc.VectorSubcoreMesh(core_axis_name="c", subcore_axis_name="s")


@functools.partial(
    pl.kernel,
    mesh=_mesh,
    out_type=jax.ShapeDtypeStruct((N_EDGES, D_MODEL), jnp.float32),
    scratch_types=(
        [pltpu.VMEM((_CH,), jnp.int32) for _ in range(_NBUF)]
        + [pltpu.VMEM((_CH, D_MODEL), jnp.float32) for _ in range(_NBUF)]
        + [pltpu.VMEM_SHARED((MAX_LEN, D_MODEL), jnp.float32)]
        + [pltpu.SemaphoreType.DMA for _ in range(3 * _NBUF)]
    ),
)
def _pe_gather(idx_hbm, table_hbm, out_hbm, *refs):
    idxs = refs[0:_NBUF]
    rowss = refs[_NBUF:2 * _NBUF]
    table_v = refs[2 * _NBUF]
    si = refs[2 * _NBUF + 1: 2 * _NBUF + 1 + _NBUF]
    sg = refs[2 * _NBUF + 1 + _NBUF: 2 * _NBUF + 1 + 2 * _NBUF]
    sw = refs[2 * _NBUF + 1 + 2 * _NBUF: 2 * _NBUF + 1 + 3 * _NBUF]

    wid = lax.axis_index("s") * _NUM_CORES + lax.axis_index("c")
    base = wid * _B_PER_W

    @pl.when(lax.axis_index("s") == 0)
    def _stage_table():
        pltpu.sync_copy(table_hbm, table_v)

    plsc.subcore_barrier()

    def fire_idx(slot, off):
        pltpu.async_copy(idx_hbm.at[pl.ds(off, _CH)], idxs[slot], si[slot])

    def wait_idx(slot, off):
        pltpu.make_async_copy(
            idx_hbm.at[pl.ds(off, _CH)], idxs[slot], si[slot]).wait()

    def fire_gather(slot):
        pltpu.async_copy(table_v.at[idxs[slot]], rowss[slot], sg[slot])

    def wait_gather(slot):
        pltpu.make_async_copy(
            table_v.at[idxs[slot]], rowss[slot], sg[slot]).wait()

    def fire_write(slot, off):
        pltpu.async_copy(rowss[slot], out_hbm.at[pl.ds(off, _CH)], sw[slot])

    def wait_write(slot, off):
        pltpu.make_async_copy(
            rowss[slot], out_hbm.at[pl.ds(off, _CH)], sw[slot]).wait()

    for b in range(_NBUF):
        fire_idx(b, base + b * _CH)

    for c in range(_LAG):
        wait_idx(c, base + c * _CH)
        fire_gather(c)
    for c in range(_LAG, _NBUF):
        b, bp = c % _NBUF, (c - _LAG) % _NBUF
        wait_idx(b, base + c * _CH)
        fire_gather(b)
        wait_gather(bp)
        fire_write(bp, base + (c - _LAG) * _CH)
        fire_idx(bp, base + (c + _NBUF - _LAG) * _CH)

    def round_(k, carry):
        for b in range(_NBUF):
            bp = (b - _LAG) % _NBUF
            c = k * _NBUF + b
            off = base + c * _CH
            wait_idx(b, off)
            wait_write(b, off - _NBUF * _CH)
            fire_gather(b)
            wait_gather(bp)
            fire_write(bp, off - _LAG * _CH)
            p = jnp.where(c + _NBUF - _LAG < _NCH, c + _NBUF - _LAG, c - _LAG)
            fire_idx(bp, base + p * _CH)
        return carry

    lax.fori_loop(1, _NCH // _NBUF, round_, 0)

    for c in range(_NCH - _LAG, _NCH):
        b = c % _NBUF
        wait_gather(b)
        fire_write(b, base + c * _CH)
    for c in range(_NCH - _NBUF, _NCH):
        b = c % _NBUF
        wait_write(b, base + c * _CH)
    for c in range(_NCH - _NBUF + _LAG, _NCH):
        bp = (c - _LAG) % _NBUF
        wait_idx(bp, base + (c - _LAG) * _CH)


def kernel(edge_type, pe):
    return _pe_gather(edge_type.astype(jnp.int32), pe)
